# Initial kernel scaffold; baseline (speedup 1.0000x reference)
import jax
import jax.numpy as jnp
from jax.experimental import pallas as pl


def kernel(continuous_value, categorical_index, cross_feat_index, cate_weights, cont_W, cont_b, wide_bias, emb_table, W1, b1, W2, b2, W3, b3):
    raise NotImplementedError("write your pallas kernel here")


# speedup vs baseline: 15.6505x; 15.6505x over previous
import functools

import jax
import jax.numpy as jnp
from jax import lax
from jax.experimental import pallas as pl
from jax.experimental.pallas import tpu as pltpu
from jax.experimental.pallas import tpu_sc as plsc

B = 16384
V = 1000000
D = 16
NCATE = 26
NCONT = 13

NC = 2
NS = 16
NW = NC * NS
SPW = B // NW
RPW = SPW * NCATE
CHUNK = 128
NCHUNK = RPW // CHUNK
GROUP = 13
OUTER = NCHUNK // GROUP
GROWS = GROUP * CHUNK

_mesh = plsc.VectorSubcoreMesh(core_axis_name="c", subcore_axis_name="s")


@functools.partial(
    pl.kernel,
    mesh=_mesh,
    compiler_params=pltpu.CompilerParams(use_tc_tiling_on_sc=False),
    out_type=[
        jax.ShapeDtypeStruct((B * NCATE, D), jnp.float32),
        jax.ShapeDtypeStruct((B,), jnp.float32),
    ],
    scratch_types=[
        pltpu.VMEM((NCHUNK, CHUNK), jnp.int32),
        pltpu.VMEM((NCHUNK, CHUNK), jnp.int32),
        pltpu.VMEM((GROWS, D), jnp.float32),
        pltpu.VMEM((RPW,), jnp.float32),
        pltpu.VMEM((SPW,), jnp.float32),
        pltpu.SemaphoreType.DMA,
        pltpu.SemaphoreType.DMA,
    ],
)
def _sc_gather(emb_hbm, w_hbm, idx_hbm, widx_hbm, out_hbm, wide_hbm,
               idx_v, widx_v, rows_v, wvals_v, wide_v, sem, wsem):
    wid = lax.axis_index("s") * NC + lax.axis_index("c")

    pltpu.sync_copy(idx_hbm.at[wid], idx_v)
    pltpu.sync_copy(widx_hbm.at[wid], widx_v)

    def _fire_w(o):
        def body(j, _):
            c = o * GROUP + j
            pltpu.async_copy(w_hbm.at[widx_v.at[c]],
                             wvals_v.at[pl.ds(c * CHUNK, CHUNK)], wsem)
            return 0
        lax.fori_loop(0, GROUP, body, 0)

    def _drain_w(o):
        pltpu.make_async_copy(w_hbm.at[pl.ds(0, GROWS)],
                              wvals_v.at[pl.ds(o * GROWS, GROWS)], wsem).wait()

    _fire_w(0)
    for o in range(1, OUTER):
        _fire_w(o)
        _drain_w(o - 1)
    _drain_w(OUTER - 1)

    def _reduce(sblk, _):
        def body(f, acc):
            return acc + wvals_v[pl.ds(f * SPW + sblk * 16, 16)]
        acc = lax.fori_loop(0, NCATE, body, jnp.zeros((16,), jnp.float32))
        wide_v[pl.ds(sblk * 16, 16)] = acc
        return 0
    lax.fori_loop(0, SPW // 16, _reduce, 0)
    pltpu.sync_copy(wide_v, wide_hbm.at[pl.ds(wid * SPW, SPW)])

    for o in range(OUTER):
        def fire(j, _):
            c = o * GROUP + j
            pltpu.async_copy(emb_hbm.at[idx_v.at[c]],
                             rows_v.at[pl.ds(j * CHUNK, CHUNK)], sem)
            return 0
        lax.fori_loop(0, GROUP, fire, 0)
        pltpu.make_async_copy(emb_hbm.at[pl.ds(0, GROWS)], rows_v, sem).wait()
        pltpu.sync_copy(rows_v,
                        out_hbm.at[pl.ds(wid * RPW + o * GROWS, GROWS)])


TB = 2048


def _mlp_body(cont_ref, emb_ref, wide_ref, w1c_ref, w1e_ref, b1_ref,
              w2_ref, b2_ref, w3_ref, b3_ref, cw_ref, wb_ref, out_ref):
    cont = cont_ref[...]
    emb = emb_ref[...]
    h = jnp.dot(emb, w1e_ref[...], preferred_element_type=jnp.float32)
    h = h + jnp.dot(cont, w1c_ref[...], preferred_element_type=jnp.float32)
    h = jnp.maximum(h + b1_ref[...], 0.0)
    h = jnp.maximum(jnp.dot(h, w2_ref[...],
                            preferred_element_type=jnp.float32) + b2_ref[...], 0.0)
    y_deep = jnp.sum(h * w3_ref[...], axis=1, keepdims=True) + b3_ref[...]
    y_wide = wide_ref[...] + jnp.sum(cont * cw_ref[...], axis=1, keepdims=True)
    y_wide = y_wide + wb_ref[...]
    out_ref[...] = jax.nn.sigmoid(y_deep + y_wide)


def _mlp_call(cont_p, emb_g, wide_c, w1c, w1e, b1, w2, b2, w3, b3, cw, wb):
    grid = (B // TB,)
    full = lambda s: pl.BlockSpec(s, lambda i: (0, 0))
    return pl.pallas_call(
        _mlp_body,
        grid=grid,
        in_specs=[
            pl.BlockSpec((TB, 16), lambda i: (i, 0)),
            pl.BlockSpec((TB, NCATE * D), lambda i: (i, 0)),
            pl.BlockSpec((TB, 1), lambda i: (i, 0)),
            full((16, 32)),
            full((NCATE * D, 32)),
            full((1, 32)),
            full((32, 32)),
            full((1, 32)),
            full((1, 32)),
            full((1, 1)),
            full((1, 16)),
            full((1, 1)),
        ],
        out_specs=pl.BlockSpec((TB, 1), lambda i: (i, 0)),
        out_shape=jax.ShapeDtypeStruct((B, 1), jnp.float32),
    )(cont_p, emb_g, wide_c, w1c, w1e, b1, w2, b2, w3, b3, cw, wb)


def kernel(continuous_value, categorical_index, cross_feat_index, cate_weights,
           cont_W, cont_b, wide_bias, emb_table, W1, b1, W2, b2, W3, b3):
    idx = categorical_index.astype(jnp.int32)
    idx_sm = idx.reshape(NW, NCHUNK, CHUNK)
    idx_fm = idx.reshape(NW, SPW, NCATE).transpose(0, 2, 1).reshape(NW, NCHUNK, CHUNK)

    emb_g, wide_c = _sc_gather(emb_table, cate_weights.reshape(V), idx_sm, idx_fm)

    cont_p = jnp.pad(continuous_value, ((0, 0), (0, 16 - NCONT)))
    cw = jnp.pad(cont_W, ((0, 0), (0, 16 - NCONT)))
    wb = (cont_b + wide_bias).reshape(1, 1)
    y = _mlp_call(
        cont_p,
        emb_g.reshape(B, NCATE * D),
        wide_c.reshape(B, 1),
        jnp.pad(W1[:, :NCONT].T, ((0, 16 - NCONT), (0, 0))),
        W1[:, NCONT:].T,
        b1.reshape(1, 32),
        W2.T,
        b2.reshape(1, 32),
        W3.reshape(1, 32),
        b3.reshape(1, 1),
        cw,
        wb,
    )
    return y
